# Initial kernel scaffold; baseline (speedup 1.0000x reference)
import jax
import jax.numpy as jnp
from jax.experimental import pallas as pl


def kernel(adj, theta):
    raise NotImplementedError("write your pallas kernel here")


# speedup vs baseline: 1.2689x; 1.2689x over previous
import jax
import jax.numpy as jnp
from jax.experimental import pallas as pl


_BLOCK_ROWS = 256


def _nlw_block(adj_ref, theta_ref, out_ref):
    b = pl.program_id(0)
    blk = adj_ref[...]
    br, n = blk.shape
    row_ids = b * br + jax.lax.broadcasted_iota(jnp.int32, (br, n), 0)
    col_ids = jax.lax.broadcasted_iota(jnp.int32, (br, n), 1)
    is_diag = row_ids == col_ids
    off_diag = jnp.where(is_diag, jnp.float32(0.0), blk)
    nbcnt = jnp.sum(off_diag, axis=1, keepdims=True) + jnp.float32(1e-10)
    theta = theta_ref[...]
    nb_w = theta / nbcnt
    out_ref[...] = jnp.where(
        is_diag, theta, jnp.where(blk != 0, nb_w, jnp.float32(0.0))
    )


def kernel(adj, theta):
    n = adj.shape[0]
    br = _BLOCK_ROWS
    grid = (n // br,)
    return pl.pallas_call(
        _nlw_block,
        grid=grid,
        in_specs=[
            pl.BlockSpec((br, n), lambda b: (b, 0)),
            pl.BlockSpec((br, 1), lambda b: (b, 0)),
        ],
        out_specs=pl.BlockSpec((br, n), lambda b: (b, 0)),
        out_shape=jax.ShapeDtypeStruct((n, n), jnp.float32),
    )(adj, theta)
